# Initial kernel scaffold; baseline (speedup 1.0000x reference)
import jax
import jax.numpy as jnp
from jax.experimental import pallas as pl


def kernel(batch_vectors, batch_keys_id, weight, cluster_size, embed_avg, hist):
    raise NotImplementedError("write your pallas kernel here")


# speedup vs baseline: 5.4158x; 5.4158x over previous
import jax
import jax.numpy as jnp
from jax.experimental import pallas as pl
from jax.experimental.pallas import tpu as pltpu

B, D, K, C = 2048, 64, 512, 100
GAMMA = 0.99
EPS = 1e-05


def _tc_body(x_ref, keys_ref, w_ref, cs_ref, ea_ref, hist_ref,
             am_ref, res_ref, wn_ref, csn_ref, ean_ref, hn_ref):
    x = x_ref[...]
    w = w_ref[...]

    xw = jax.lax.dot_general(x, w, (((1,), (0,)), ((), ())),
                             precision=jax.lax.Precision.HIGHEST)
    x2 = jnp.sum(x * x, axis=1, keepdims=True)
    w2 = jnp.sum(w * w, axis=0, keepdims=True)
    d2 = jnp.maximum(x2 - 2.0 * xw + w2, 0.0)
    dist = jnp.sqrt(d2)
    am = jnp.argmin(dist, axis=1)
    am_col = am[:, None]
    am_ref[...] = am_col

    onehot = (am_col == jax.lax.broadcasted_iota(jnp.int32, (1, K), 1)
              ).astype(jnp.float32)

    res_ref[...] = jax.lax.dot_general(
        onehot, w, (((1,), (1,)), ((), ())),
        precision=jax.lax.Precision.HIGHEST)

    n_idx = jnp.sum(onehot, axis=0, keepdims=True)
    n_idx = jnp.where(n_idx == 0.0, 1.0, n_idx)
    cs_new = cs_ref[...] * GAMMA + (1.0 - GAMMA) * n_idx
    csn_ref[...] = cs_new

    embed_sum = jax.lax.dot_general(
        x, onehot, (((0,), (0,)), ((), ())),
        precision=jax.lax.Precision.HIGHEST)
    ea_new = ea_ref[...] * GAMMA + (1.0 - GAMMA) * embed_sum
    ean_ref[...] = ea_new

    n = jnp.sum(cs_new)
    cs_smoothed = (cs_new + EPS) / (n + K * EPS) * n
    wn_ref[...] = ea_new / cs_smoothed

    keys_oh = (keys_ref[...] == jax.lax.broadcasted_iota(jnp.int32, (1, C), 1)
               ).astype(jnp.float32)
    bh = jax.lax.dot_general(
        onehot, keys_oh, (((0,), (0,)), ((), ())),
        precision=jax.lax.Precision.HIGHEST)
    hn_ref[...] = hist_ref[...] * GAMMA + (1.0 - GAMMA) * bh


def kernel(batch_vectors, batch_keys_id, weight, cluster_size, embed_avg, hist):
    keys2d = batch_keys_id.reshape(B, 1).astype(jnp.int32)
    cs2d = cluster_size.reshape(1, K)
    out_shapes = (
        jax.ShapeDtypeStruct((B, 1), jnp.int32),
        jax.ShapeDtypeStruct((B, D), jnp.float32),
        jax.ShapeDtypeStruct((D, K), jnp.float32),
        jax.ShapeDtypeStruct((1, K), jnp.float32),
        jax.ShapeDtypeStruct((D, K), jnp.float32),
        jax.ShapeDtypeStruct((K, C), jnp.float32),
    )
    am, result, weight_new, cs_new, ea_new, hist_new = pl.pallas_call(
        _tc_body,
        out_shape=out_shapes,
    )(batch_vectors, keys2d, weight, cs2d, embed_avg, hist)
    return (result, am.reshape(B), weight_new, cs_new.reshape(K), ea_new,
            hist_new)
